# baseline (speedup 1.0000x reference)
import functools

import jax
import jax.numpy as jnp
from jax import lax
from jax.experimental import pallas as pl
from jax.experimental.pallas import tpu as pltpu
from jax.experimental.pallas import tpu_sc as plsc

B = 16384
V = 1000000
E = 64
CTX = 5
H = 256
OUT = 50

N = B * CTX
P = 2 * E
K = 499712
VP = 500288


@functools.lru_cache(maxsize=1)
def _make_sc_gather():
    info = plsc.get_sparse_core_info()
    NC, NS = info.num_cores, info.num_subcores
    NW = NC * NS
    n_per_w = N // NW
    CH = 320
    n_ch = n_per_w // CH

    mesh = plsc.VectorSubcoreMesh(core_axis_name="c", subcore_axis_name="s")

    @functools.partial(
        pl.kernel,
        mesh=mesh,
        out_type=jax.ShapeDtypeStruct((N, P), jnp.float32),
        scratch_types=[
            pltpu.VMEM((n_per_w,), jnp.int32),
            pltpu.VMEM((CH, P), jnp.float32),
            pltpu.VMEM((CH, P), jnp.float32),
            pltpu.SemaphoreType.DMA,
            pltpu.SemaphoreType.DMA,
        ],
    )
    def gather_k(table_hbm, idx_hbm, out_hbm, idx_v, buf0, buf1, sem0, sem1):
        wid = lax.axis_index("s") * NC + lax.axis_index("c")
        base = wid * n_per_w
        pltpu.sync_copy(idx_hbm.at[pl.ds(base, n_per_w)], idx_v)
        bufs = (buf0, buf1)
        sems = (sem0, sem1)
        copies = [None, None]
        for ci in range(n_ch):
            s = ci % 2
            copies[s] = pltpu.async_copy(
                table_hbm.at[idx_v.at[pl.ds(ci * CH, CH)]], bufs[s], sems[s]
            )
            if ci > 0:
                p = (ci - 1) % 2
                copies[p].wait()
                pltpu.sync_copy(bufs[p], out_hbm.at[pl.ds(base + (ci - 1) * CH, CH)])
        last = (n_ch - 1) % 2
        copies[last].wait()
        pltpu.sync_copy(bufs[last], out_hbm.at[pl.ds(base + (n_ch - 1) * CH, CH)])

    return gather_k


_TPB = 8192


def _tr_body(tl_ref, tr_ref, out_ref):
    ri = lax.broadcasted_iota(jnp.int32, (E, E), 0)
    ci = lax.broadcasted_iota(jnp.int32, (E, E), 1)
    eye = (ri == ci).astype(jnp.float32)
    ttl = lax.dot_general(
        tl_ref[...], eye, (((0,), (0,)), ((), ())),
        preferred_element_type=jnp.float32,
    )
    ttr = lax.dot_general(
        tr_ref[...], eye, (((0,), (0,)), ((), ())),
        preferred_element_type=jnp.float32,
    )
    out_ref[...] = jnp.concatenate([ttl, ttr], axis=1)


def _transpose_pack(tT):
    grid = ((VP + _TPB - 1) // _TPB,)
    return pl.pallas_call(
        _tr_body,
        grid=grid,
        in_specs=[
            pl.BlockSpec((E, _TPB), lambda i: (0, i)),
            pl.BlockSpec((E, _TPB), lambda i: (0, i + K // _TPB)),
        ],
        out_specs=pl.BlockSpec((_TPB, P), lambda i: (i, 0)),
        out_shape=jax.ShapeDtypeStruct((VP, P), jnp.float32),
    )(tT, tT)


_BLK = 2048


def _mlp_body(emb_ref, x_ref, w1d_ref, b1_ref, w2_ref, b2_ref, out_ref):
    x = x_ref[...]
    hi = (x >= K).astype(jnp.float32)
    nz = (x != 0).astype(jnp.float32)
    w_lo = nz * (1.0 - hi)
    padb = jnp.concatenate([w_lo, hi], axis=1)
    ri = lax.broadcasted_iota(jnp.int32, (2 * CTX, P), 0)
    ci = lax.broadcasted_iota(jnp.int32, (2 * CTX, P), 1)
    acc = b1_ref[...].astype(jnp.float32)
    for c in range(CTX):
        S_c = (
            ((ri == c) & (ci < E)) | ((ri == CTX + c) & (ci >= E))
        ).astype(jnp.float32)
        mask_c = jnp.dot(padb, S_c, preferred_element_type=jnp.float32)
        acc = acc + jnp.dot(
            emb_ref[:, c, :] * mask_c,
            w1d_ref[c],
            preferred_element_type=jnp.float32,
        )
    h = jnp.tanh(acc)
    out_ref[...] = (
        jnp.dot(h, w2_ref[...], preferred_element_type=jnp.float32) + b2_ref[...]
    )


def _mlp(emb, x32, W1dup, b1, W2, b2):
    grid = (B // _BLK,)
    return pl.pallas_call(
        _mlp_body,
        grid=grid,
        in_specs=[
            pl.BlockSpec((_BLK, CTX, P), lambda i: (i, 0, 0)),
            pl.BlockSpec((_BLK, CTX), lambda i: (i, 0)),
            pl.BlockSpec((CTX, P, H), lambda i: (0, 0, 0)),
            pl.BlockSpec((1, H), lambda i: (0, 0)),
            pl.BlockSpec((H, OUT), lambda i: (0, 0)),
            pl.BlockSpec((1, OUT), lambda i: (0, 0)),
        ],
        out_specs=pl.BlockSpec((_BLK, OUT), lambda i: (i, 0)),
        out_shape=jax.ShapeDtypeStruct((B, OUT), jnp.float32),
    )(emb, x32, W1dup, b1, W2, b2)


def kernel(x, table, W1, b1, W2, b2):
    x32 = x.astype(jnp.int32)
    idx = x32.reshape(-1)
    pidx = jnp.where(idx < K, idx, idx - K)
    tp = _transpose_pack(table.T)
    rows = _make_sc_gather()(tp, pidx)
    emb = rows.reshape(B, CTX, P)
    w1r = W1.reshape(CTX, E, H)
    W1dup = jnp.concatenate([w1r, w1r], axis=1)
    out = _mlp(emb, x32, W1dup, b1.reshape(1, H), W2, b2.reshape(1, OUT))
    return out


# speedup vs baseline: 1.2940x; 1.2940x over previous
import functools

import jax
import jax.numpy as jnp
from jax import lax
from jax.experimental import pallas as pl
from jax.experimental.pallas import tpu as pltpu
from jax.experimental.pallas import tpu_sc as plsc

B = 16384
V = 1000000
E = 64
CTX = 5
H = 256
OUT = 50

N = B * CTX
P = 2 * E
K = 499712
VP = 500288


@functools.lru_cache(maxsize=1)
def _make_sc_gather():
    info = plsc.get_sparse_core_info()
    NC, NS = info.num_cores, info.num_subcores
    NW = NC * NS
    b_per_w = B // NW
    CHR = 64
    n_ch = b_per_w // CHR

    mesh = plsc.VectorSubcoreMesh(core_axis_name="c", subcore_axis_name="s")

    @functools.partial(
        pl.kernel,
        mesh=mesh,
        out_type=jax.ShapeDtypeStruct((B, CTX * P), jnp.float32),
        scratch_types=[
            pltpu.VMEM((CTX, b_per_w), jnp.int32),
            pltpu.VMEM((CHR, CTX * P), jnp.float32),
            pltpu.VMEM((CHR, CTX * P), jnp.float32),
            pltpu.SemaphoreType.DMA,
            pltpu.SemaphoreType.DMA,
        ],
    )
    def gather_k(table_hbm, idx_hbm, out_hbm, idx_v, buf0, buf1, sem0, sem1):
        wid = lax.axis_index("s") * NC + lax.axis_index("c")
        base = wid * b_per_w
        pltpu.sync_copy(idx_hbm.at[:, pl.ds(base, b_per_w)], idx_v)
        bufs = (buf0, buf1)
        sems = (sem0, sem1)
        copies = [[None] * CTX, [None] * CTX]
        for ci in range(n_ch):
            s = ci % 2
            for c in range(CTX):
                copies[s][c] = pltpu.async_copy(
                    table_hbm.at[idx_v.at[c, pl.ds(ci * CHR, CHR)]],
                    bufs[s].at[:, pl.ds(c * P, P)],
                    sems[s],
                )
            if ci > 0:
                pv = (ci - 1) % 2
                for c in range(CTX):
                    copies[pv][c].wait()
                pltpu.sync_copy(
                    bufs[pv], out_hbm.at[pl.ds(base + (ci - 1) * CHR, CHR)]
                )
        last = (n_ch - 1) % 2
        for c in range(CTX):
            copies[last][c].wait()
        pltpu.sync_copy(bufs[last], out_hbm.at[pl.ds(base + (n_ch - 1) * CHR, CHR)])

    return gather_k


_TPB = 8192


def _tr_body(tl_ref, tr_ref, out_ref):
    ri = lax.broadcasted_iota(jnp.int32, (E, E), 0)
    ci = lax.broadcasted_iota(jnp.int32, (E, E), 1)
    eye = (ri == ci).astype(jnp.float32)
    ttl = lax.dot_general(
        tl_ref[...], eye, (((0,), (0,)), ((), ())),
        preferred_element_type=jnp.float32,
    )
    ttr = lax.dot_general(
        tr_ref[...], eye, (((0,), (0,)), ((), ())),
        preferred_element_type=jnp.float32,
    )
    out_ref[...] = jnp.concatenate([ttl, ttr], axis=1)


def _transpose_pack(tT):
    grid = ((VP + _TPB - 1) // _TPB,)
    return pl.pallas_call(
        _tr_body,
        grid=grid,
        in_specs=[
            pl.BlockSpec((E, _TPB), lambda i: (0, i)),
            pl.BlockSpec((E, _TPB), lambda i: (0, i + K // _TPB)),
        ],
        out_specs=pl.BlockSpec((_TPB, P), lambda i: (i, 0)),
        out_shape=jax.ShapeDtypeStruct((VP, P), jnp.float32),
    )(tT, tT)


_BLK = 2048


def _mlp_body(flat_ref, x_ref, w1d_ref, b1_ref, w2_ref, b2_ref, out_ref):
    x = x_ref[...]
    hi = (x >= K).astype(jnp.float32)
    nz = (x != 0).astype(jnp.float32)
    w_lo = nz * (1.0 - hi)
    padb = jnp.concatenate([w_lo, hi], axis=1)
    ri = lax.broadcasted_iota(jnp.int32, (2 * CTX, CTX * P), 0)
    ci = lax.broadcasted_iota(jnp.int32, (2 * CTX, CTX * P), 1)
    grp = ci // E
    sel = jnp.where(ri < CTX, 2 * ri, 2 * (ri - CTX) + 1)
    S = (grp == sel).astype(jnp.float32)
    mask = jnp.dot(padb, S, preferred_element_type=jnp.float32)
    acc = jnp.dot(
        flat_ref[...] * mask, w1d_ref[...], preferred_element_type=jnp.float32
    )
    h = jnp.tanh(acc + b1_ref[...])
    out_ref[...] = (
        jnp.dot(h, w2_ref[...], preferred_element_type=jnp.float32) + b2_ref[...]
    )


def _mlp(flat, x32, W1dup, b1, W2, b2):
    grid = (B // _BLK,)
    return pl.pallas_call(
        _mlp_body,
        grid=grid,
        in_specs=[
            pl.BlockSpec((_BLK, CTX * P), lambda i: (i, 0)),
            pl.BlockSpec((_BLK, CTX), lambda i: (i, 0)),
            pl.BlockSpec((CTX * P, H), lambda i: (0, 0)),
            pl.BlockSpec((1, H), lambda i: (0, 0)),
            pl.BlockSpec((H, OUT), lambda i: (0, 0)),
            pl.BlockSpec((1, OUT), lambda i: (0, 0)),
        ],
        out_specs=pl.BlockSpec((_BLK, OUT), lambda i: (i, 0)),
        out_shape=jax.ShapeDtypeStruct((B, OUT), jnp.float32),
    )(flat, x32, W1dup, b1, W2, b2)


def kernel(x, table, W1, b1, W2, b2):
    x32 = x.astype(jnp.int32)
    xT = x32.T
    pidxT = jnp.where(xT < K, xT, xT - K)
    tp = _transpose_pack(table.T)
    flat = _make_sc_gather()(tp, pidxT)
    w1r = W1.reshape(CTX, E, H)
    W1dup = jnp.concatenate([w1r, w1r], axis=1).reshape(CTX * P, H)
    out = _mlp(flat, x32, W1dup, b1.reshape(1, H), W2, b2.reshape(1, OUT))
    return out


# speedup vs baseline: 1.3285x; 1.0266x over previous
import functools

import jax
import jax.numpy as jnp
from jax import lax
from jax.experimental import pallas as pl
from jax.experimental.pallas import tpu as pltpu
from jax.experimental.pallas import tpu_sc as plsc

B = 16384
V = 1000000
E = 64
CTX = 5
H = 256
OUT = 50

N = B * CTX
P = 2 * E
K = 491520
VP = 508480


@functools.lru_cache(maxsize=1)
def _make_sc_gather():
    info = plsc.get_sparse_core_info()
    NC, NS = info.num_cores, info.num_subcores
    NW = NC * NS
    b_per_w = B // NW
    CHR = 64
    n_ch = b_per_w // CHR

    mesh = plsc.VectorSubcoreMesh(core_axis_name="c", subcore_axis_name="s")

    @functools.partial(
        pl.kernel,
        mesh=mesh,
        out_type=jax.ShapeDtypeStruct((B, CTX * P), jnp.float32),
        scratch_types=[
            pltpu.VMEM((CTX, b_per_w), jnp.int32),
            pltpu.VMEM((CHR, CTX * P), jnp.float32),
            pltpu.VMEM((CHR, CTX * P), jnp.float32),
            pltpu.SemaphoreType.DMA,
            pltpu.SemaphoreType.DMA,
        ],
    )
    def gather_k(table_hbm, idx_hbm, out_hbm, idx_v, buf0, buf1, sem0, sem1):
        wid = lax.axis_index("s") * NC + lax.axis_index("c")
        base = wid * b_per_w
        pltpu.sync_copy(idx_hbm.at[:, pl.ds(base, b_per_w)], idx_v)
        bufs = (buf0, buf1)
        sems = (sem0, sem1)
        copies = [[None] * CTX, [None] * CTX]
        for ci in range(n_ch):
            s = ci % 2
            for c in range(CTX):
                copies[s][c] = pltpu.async_copy(
                    table_hbm.at[idx_v.at[c, pl.ds(ci * CHR, CHR)]],
                    bufs[s].at[:, pl.ds(c * P, P)],
                    sems[s],
                )
            if ci > 0:
                pv = (ci - 1) % 2
                for c in range(CTX):
                    copies[pv][c].wait()
                pltpu.sync_copy(
                    bufs[pv], out_hbm.at[pl.ds(base + (ci - 1) * CHR, CHR)]
                )
        last = (n_ch - 1) % 2
        for c in range(CTX):
            copies[last][c].wait()
        pltpu.sync_copy(bufs[last], out_hbm.at[pl.ds(base + (n_ch - 1) * CHR, CHR)])

    return gather_k


_TPB = 16384


def _tr_body(tl_ref, tr_ref, out_ref):
    ri = lax.broadcasted_iota(jnp.int32, (E, E), 0)
    ci = lax.broadcasted_iota(jnp.int32, (E, E), 1)
    eye = (ri == ci).astype(jnp.float32)
    ttl = lax.dot_general(
        tl_ref[...], eye, (((0,), (0,)), ((), ())),
        preferred_element_type=jnp.float32,
    )
    ttr = lax.dot_general(
        tr_ref[...], eye, (((0,), (0,)), ((), ())),
        preferred_element_type=jnp.float32,
    )
    out_ref[...] = jnp.concatenate([ttl, ttr], axis=1)


def _transpose_pack(tT):
    grid = ((VP + _TPB - 1) // _TPB,)
    return pl.pallas_call(
        _tr_body,
        grid=grid,
        in_specs=[
            pl.BlockSpec((E, _TPB), lambda i: (0, i)),
            pl.BlockSpec((E, _TPB), lambda i: (0, i + K // _TPB)),
        ],
        out_specs=pl.BlockSpec((_TPB, P), lambda i: (i, 0)),
        out_shape=jax.ShapeDtypeStruct((VP, P), jnp.float32),
    )(tT, tT)


_BLK = 2048


def _mlp_body(flat_ref, x_ref, w1d_ref, b1_ref, w2_ref, b2_ref, out_ref):
    x = x_ref[...]
    hi = (x >= K).astype(jnp.float32)
    nz = (x != 0).astype(jnp.float32)
    w_lo = nz * (1.0 - hi)
    padb = jnp.concatenate([w_lo, hi], axis=1)
    ri = lax.broadcasted_iota(jnp.int32, (2 * CTX, CTX * P), 0)
    ci = lax.broadcasted_iota(jnp.int32, (2 * CTX, CTX * P), 1)
    grp = ci // E
    sel = jnp.where(ri < CTX, 2 * ri, 2 * (ri - CTX) + 1)
    S = (grp == sel).astype(jnp.float32)
    mask = jnp.dot(padb, S, preferred_element_type=jnp.float32)
    acc = jnp.dot(
        flat_ref[...] * mask, w1d_ref[...], preferred_element_type=jnp.float32
    )
    h = jnp.tanh(acc + b1_ref[...])
    out_ref[...] = (
        jnp.dot(h, w2_ref[...], preferred_element_type=jnp.float32) + b2_ref[...]
    )


def _mlp(flat, x32, W1dup, b1, W2, b2):
    grid = (B // _BLK,)
    return pl.pallas_call(
        _mlp_body,
        grid=grid,
        in_specs=[
            pl.BlockSpec((_BLK, CTX * P), lambda i: (i, 0)),
            pl.BlockSpec((_BLK, CTX), lambda i: (i, 0)),
            pl.BlockSpec((CTX * P, H), lambda i: (0, 0)),
            pl.BlockSpec((1, H), lambda i: (0, 0)),
            pl.BlockSpec((H, OUT), lambda i: (0, 0)),
            pl.BlockSpec((1, OUT), lambda i: (0, 0)),
        ],
        out_specs=pl.BlockSpec((_BLK, OUT), lambda i: (i, 0)),
        out_shape=jax.ShapeDtypeStruct((B, OUT), jnp.float32),
    )(flat, x32, W1dup, b1, W2, b2)


def kernel(x, table, W1, b1, W2, b2):
    x32 = x.astype(jnp.int32)
    xT = x32.T
    pidxT = jnp.where(xT < K, xT, xT - K)
    tp = _transpose_pack(table.T)
    flat = _make_sc_gather()(tp, pidxT)
    w1r = W1.reshape(CTX, E, H)
    W1dup = jnp.concatenate([w1r, w1r], axis=1).reshape(CTX * P, H)
    out = _mlp(flat, x32, W1dup, b1.reshape(1, H), W2, b2.reshape(1, OUT))
    return out


# speedup vs baseline: 1.4316x; 1.0777x over previous
import functools

import jax
import jax.numpy as jnp
from jax import lax
from jax.experimental import pallas as pl
from jax.experimental.pallas import tpu as pltpu
from jax.experimental.pallas import tpu_sc as plsc

B = 16384
V = 1000000
E = 64
CTX = 5
H = 256
OUT = 50

N = B * CTX
P = 2 * E
_TPB = 8192
KQ = 245760
VQ = 262720


@functools.lru_cache(maxsize=1)
def _make_sc_gather():
    info = plsc.get_sparse_core_info()
    NC, NS = info.num_cores, info.num_subcores
    NW = NC * NS
    b_per_w = B // NW
    CHR = 64
    n_ch = b_per_w // CHR

    mesh = plsc.VectorSubcoreMesh(core_axis_name="c", subcore_axis_name="s")

    @functools.partial(
        pl.kernel,
        mesh=mesh,
        out_type=jax.ShapeDtypeStruct((B, CTX * P), jnp.int32),
        scratch_types=[
            pltpu.VMEM((CTX, b_per_w), jnp.int32),
            pltpu.VMEM((CHR, CTX * P), jnp.int32),
            pltpu.VMEM((CHR, CTX * P), jnp.int32),
            pltpu.SemaphoreType.DMA,
            pltpu.SemaphoreType.DMA,
        ],
    )
    def gather_k(table_hbm, idx_hbm, out_hbm, idx_v, buf0, buf1, sem0, sem1):
        wid = lax.axis_index("s") * NC + lax.axis_index("c")
        base = wid * b_per_w
        pltpu.sync_copy(idx_hbm.at[:, pl.ds(base, b_per_w)], idx_v)
        bufs = (buf0, buf1)
        sems = (sem0, sem1)
        copies = [[None] * CTX, [None] * CTX]
        for ci in range(n_ch):
            s = ci % 2
            for c in range(CTX):
                copies[s][c] = pltpu.async_copy(
                    table_hbm.at[idx_v.at[c, pl.ds(ci * CHR, CHR)]],
                    bufs[s].at[:, pl.ds(c * P, P)],
                    sems[s],
                )
            if ci > 0:
                pv = (ci - 1) % 2
                for c in range(CTX):
                    copies[pv][c].wait()
                pltpu.sync_copy(
                    bufs[pv], out_hbm.at[pl.ds(base + (ci - 1) * CHR, CHR)]
                )
        last = (n_ch - 1) % 2
        for c in range(CTX):
            copies[last][c].wait()
        pltpu.sync_copy(bufs[last], out_hbm.at[pl.ds(base + (n_ch - 1) * CHR, CHR)])

    return gather_k


def _bfbits(v):
    return lax.bitcast_convert_type(
        v.astype(jnp.bfloat16).astype(jnp.float32), jnp.uint32
    )


def _tr_body(t0_ref, t1_ref, t2_ref, t3_ref, out_ref):
    ri = lax.broadcasted_iota(jnp.int32, (E, E), 0)
    ci = lax.broadcasted_iota(jnp.int32, (E, E), 1)
    eye = (ri == ci).astype(jnp.float32)

    def tr(ref):
        return lax.dot_general(
            ref[...], eye, (((0,), (0,)), ((), ())),
            preferred_element_type=jnp.float32,
        )

    tt0, tt1, tt2, tt3 = tr(t0_ref), tr(t1_ref), tr(t2_ref), tr(t3_ref)
    sh = jnp.uint32(16)
    lo = _bfbits(tt0) | jnp.right_shift(_bfbits(tt1), sh)
    hi = _bfbits(tt2) | jnp.right_shift(_bfbits(tt3), sh)
    out_ref[...] = lax.bitcast_convert_type(
        jnp.concatenate([lo, hi], axis=1), jnp.int32
    )


def _transpose_pack(tT):
    grid = ((VQ + _TPB - 1) // _TPB,)
    nb = KQ // _TPB
    return pl.pallas_call(
        _tr_body,
        grid=grid,
        in_specs=[
            pl.BlockSpec((E, _TPB), lambda i: (0, i)),
            pl.BlockSpec((E, _TPB), lambda i: (0, i + nb)),
            pl.BlockSpec((E, _TPB), lambda i: (0, i + 2 * nb)),
            pl.BlockSpec((E, _TPB), lambda i: (0, i + 3 * nb)),
        ],
        out_specs=pl.BlockSpec((_TPB, P), lambda i: (i, 0)),
        out_shape=jax.ShapeDtypeStruct((VQ, P), jnp.int32),
    )(tT, tT, tT, tT)


_BLK = 2048


def _mlp_body(flat_ref, x_ref, w1d_ref, b1_ref, w2_ref, b2_ref, out_ref):
    x = x_ref[...]
    q1 = (x >= KQ).astype(jnp.float32)
    q2 = (x >= 2 * KQ).astype(jnp.float32)
    q3 = (x >= 3 * KQ).astype(jnp.float32)
    nz = (x != 0).astype(jnp.float32)
    w0 = nz * (1.0 - q1)
    w1 = q1 * (1.0 - q2)
    w2 = q2 * (1.0 - q3)
    w3 = q3
    padb = jnp.concatenate([w0, w1, w2, w3], axis=1)
    ri = lax.broadcasted_iota(jnp.int32, (4 * CTX, CTX * P), 0)
    ci = lax.broadcasted_iota(jnp.int32, (4 * CTX, CTX * P), 1)
    c_of = ci // P
    half = (ci % P) // E
    SA = (
        ((ri == c_of) & (half == 0)) | ((ri == 2 * CTX + c_of) & (half == 1))
    ).astype(jnp.float32)
    SB = (
        ((ri == CTX + c_of) & (half == 0))
        | ((ri == 3 * CTX + c_of) & (half == 1))
    ).astype(jnp.float32)
    maskA = jnp.dot(padb, SA, preferred_element_type=jnp.float32)
    maskB = jnp.dot(padb, SB, preferred_element_type=jnp.float32)
    w = flat_ref[...]
    A = lax.bitcast_convert_type(w & jnp.int32(-65536), jnp.float32)
    Bv = lax.bitcast_convert_type(jnp.left_shift(w, 16), jnp.float32)
    val = A * maskA + Bv * maskB
    acc = jnp.dot(val, w1d_ref[...], preferred_element_type=jnp.float32)
    h = jnp.tanh(acc + b1_ref[...])
    out_ref[...] = (
        jnp.dot(h, w2_ref[...], preferred_element_type=jnp.float32) + b2_ref[...]
    )


def _mlp(flat, x32, W1dup, b1, W2, b2):
    grid = (B // _BLK,)
    return pl.pallas_call(
        _mlp_body,
        grid=grid,
        in_specs=[
            pl.BlockSpec((_BLK, CTX * P), lambda i: (i, 0)),
            pl.BlockSpec((_BLK, CTX), lambda i: (i, 0)),
            pl.BlockSpec((CTX * P, H), lambda i: (0, 0)),
            pl.BlockSpec((1, H), lambda i: (0, 0)),
            pl.BlockSpec((H, OUT), lambda i: (0, 0)),
            pl.BlockSpec((1, OUT), lambda i: (0, 0)),
        ],
        out_specs=pl.BlockSpec((_BLK, OUT), lambda i: (i, 0)),
        out_shape=jax.ShapeDtypeStruct((B, OUT), jnp.float32),
    )(flat, x32, W1dup, b1, W2, b2)


def kernel(x, table, W1, b1, W2, b2):
    x32 = x.astype(jnp.int32)
    xT = x32.T
    g = (
        (xT >= KQ).astype(jnp.int32)
        + (xT >= 2 * KQ).astype(jnp.int32)
        + (xT >= 3 * KQ).astype(jnp.int32)
    )
    pidxT = xT - g * KQ
    tq = _transpose_pack(table.T)
    flat = _make_sc_gather()(tq, pidxT)
    w1r = W1.reshape(CTX, E, H)
    W1dup = jnp.concatenate([w1r, w1r], axis=1).reshape(CTX * P, H)
    out = _mlp(flat, x32, W1dup, b1.reshape(1, H), W2, b2.reshape(1, OUT))
    return out
